# baseline (speedup 1.0000x reference)
import functools

import jax
import jax.numpy as jnp
from jax import lax
from jax.experimental import pallas as pl
from jax.experimental.pallas import tpu as pltpu
from jax.experimental.pallas import tpu_sc as plsc


def _iota16():
    return lax.iota(jnp.int32, 16)


@functools.cache
def _make_transpose_kernel(V: int, D: int):
    assert D == 64
    info = plsc.get_sparse_core_info()
    NC, NS = info.num_cores, info.num_subcores
    NW = NC * NS
    BLK = 256
    n_full = V // BLK
    n_tail = V - n_full * BLK
    assert n_tail % 8 == 0
    pairs = (n_full // NW + 1 + 1) // 2

    mesh = plsc.VectorSubcoreMesh(core_axis_name="c", subcore_axis_name="s")

    @functools.partial(
        pl.kernel,
        mesh=mesh,
        compiler_params=pltpu.CompilerParams(
            use_tc_tiling_on_sc=True, needs_layout_passes=False
        ),
        out_type=jax.ShapeDtypeStruct((V // 2, 2 * D), jnp.float32),
        scratch_types=[
            pltpu.VMEM((D, BLK + 1), jnp.float32),
            pltpu.VMEM((D, BLK + 1), jnp.float32),
            pltpu.VMEM((2 * D, BLK // 2), jnp.float32),
            pltpu.VMEM((2 * D, BLK // 2), jnp.float32),
            pltpu.SemaphoreType.DMA,
            pltpu.SemaphoreType.DMA,
            pltpu.SemaphoreType.DMA,
            pltpu.SemaphoreType.DMA,
        ],
    )
    def tr_kernel(wT_hbm, tail_hbm, wp_hbm, a0, a1, t0, t1, i0, i1, o0, o1):
        wid = lax.axis_index("s") * NC + lax.axis_index("c")
        abufs = (a0, a1)
        tbufs = (t0, t1)
        isems = (i0, i1)
        osems = (o0, o1)
        iot = _iota16()

        def in_desc(blk, b):
            v0 = pl.multiple_of(blk * BLK, BLK)
            return pltpu.make_async_copy(
                wT_hbm.at[:, pl.ds(v0, BLK)],
                abufs[b].at[:, pl.ds(0, BLK)],
                isems[b],
            )

        def out_desc(blk, b):
            r0 = pl.multiple_of(blk * (BLK // 2), BLK // 2)
            return pltpu.make_async_copy(
                tbufs[b], wp_hbm.at[pl.ds(r0, BLK // 2)], osems[b]
            )

        def transpose_block(b):
            a = abufs[b]
            t = tbufs[b]

            rbase = tuple(iot + 16 * q for q in range(4))

            @plsc.parallel_loop(0, BLK // 2, unroll=4, carry=rbase)
            def krow(k, rb):
                colv0 = jnp.full((16,), 2 * k, jnp.int32)
                colv1 = colv0 + 1
                vals = [
                    plsc.load_gather(
                        a, [rb[g % 4], colv0 if g < 4 else colv1]
                    )
                    for g in range(8)
                ]
                for g in range(8):
                    t[k, pl.ds(16 * g, 16)] = vals[g]
                return rb

        in_desc(wid, 0).start()
        in_desc(wid + NW, 1).start()

        def body(t, carry):
            for half in range(2):
                blk = wid + NW * (2 * t + half)

                @pl.when(blk < n_full)
                def _():
                    in_desc(blk, half).wait()
                    transpose_block(half)

                    @pl.when(t > 0)
                    def _():
                        out_desc(0, half).wait()

                    out_desc(blk, half).start()
                    nxt = blk + 2 * NW

                    @pl.when(nxt < n_full)
                    def _():
                        in_desc(nxt, half).start()

            return carry

        lax.fori_loop(0, pairs, body, 0)
        out_desc(0, 0).wait()
        out_desc(0, 1).wait()

        @pl.when(wid == 2)
        def _():
            tail_v = tbufs[0].at[pl.ds(0, n_tail // 2)]
            pltpu.sync_copy(tail_hbm, tail_v)
            pltpu.sync_copy(
                tail_v, wp_hbm.at[pl.ds(n_full * (BLK // 2), n_tail // 2)]
            )

    return tr_kernel


@functools.cache
def _make_gather_kernel(V: int, D: int, NB: int, H: int):
    assert D == 64
    info = plsc.get_sparse_core_info()
    NC, NS = info.num_cores, info.num_subcores
    NW = NC * NS
    HG = 8
    BB = 128
    n_units = (H // HG) * (NB // BB)
    nb_blocks = NB // BB
    per_tile = (n_units + NW - 1) // NW

    mesh = plsc.VectorSubcoreMesh(core_axis_name="c", subcore_axis_name="s")

    @functools.partial(
        pl.kernel,
        mesh=mesh,
        compiler_params=pltpu.CompilerParams(
            use_tc_tiling_on_sc=True, needs_layout_passes=False
        ),
        out_type=jax.ShapeDtypeStruct((H, D, NB), jnp.float32),
        scratch_types=[
            pltpu.VMEM((HG, BB), jnp.int32),
            pltpu.VMEM((BB,), jnp.int32),
            pltpu.VMEM((BB,), jnp.int32),
            pltpu.VMEM((BB, 2 * D + 1), jnp.float32),
            pltpu.VMEM((BB, 2 * D + 1), jnp.float32),
            pltpu.VMEM((D, BB + 1), jnp.float32),
            pltpu.VMEM((D, BB + 1), jnp.float32),
            pltpu.SemaphoreType.DMA,
            pltpu.SemaphoreType.DMA,
            pltpu.SemaphoreType.DMA,
            pltpu.SemaphoreType.DMA,
        ],
    )
    def g_kernel(
        wp_hbm, xT_hbm, out_hbm, ibuf, w0, w1, r0, r1, t0, t1, g0, g1, o0, o1
    ):
        wid = lax.axis_index("s") * NC + lax.axis_index("c")
        wbufs = (w0, w1)
        rbufs = (r0, r1)
        tbufs = (t0, t1)
        gsems = (g0, g1)
        osems = (o0, o1)
        iot = _iota16()

        def compute_widx(h, hb):
            dst = wbufs[hb]

            @plsc.parallel_loop(0, BB // 16, unroll=4)
            def mbody(m):
                dst[pl.ds(m * 16, 16)] = lax.shift_right_logical(
                    ibuf[h, pl.ds(m * 16, 16)], 1
                )

        def gather_descs(hb):
            return [
                pltpu.make_async_copy(
                    wp_hbm.at[wbufs[hb].at[pl.ds(k * 128, 128)]],
                    rbufs[hb].at[pl.ds(k * 128, 128), pl.ds(0, 2 * D)],
                    gsems[hb],
                )
                for k in range(BB // 128)
            ]

        def transpose_rows(h, hb):
            rows = rbufs[hb]
            t = tbufs[hb]

            @plsc.parallel_loop(0, BB // 16, unroll=1, carry=iot)
            def mbody(m, io):
                par = lax.bitwise_and(ibuf[h, pl.ds(m * 16, 16)], 1)
                cbase = par * D
                rvec = io + m * 16

                @plsc.parallel_loop(0, D, step=8, unroll=1, carry=(rvec, cbase))
                def fbody(f, c):
                    rv, cb = c
                    vals = [
                        plsc.load_gather(rows, [rv, cb + (f + q)])
                        for q in range(8)
                    ]
                    for q in range(8):
                        t[f + q, pl.ds(m * 16, 16)] = vals[q]
                    return c

                return io

        def out_desc(hh, b0, hb):
            return pltpu.make_async_copy(
                tbufs[hb].at[:, pl.ds(0, BB)],
                out_hbm.at[hh, :, pl.ds(pl.multiple_of(b0, BB), BB)],
                osems[hb],
            )

        def unit(u):
            g = u // nb_blocks
            h0 = pl.multiple_of(g * HG, HG)
            b0 = pl.multiple_of((u - g * nb_blocks) * BB, BB)
            pltpu.sync_copy(xT_hbm.at[pl.ds(h0, HG), pl.ds(b0, BB)], ibuf)
            compute_widx(0, 0)
            for cp in gather_descs(0):
                cp.start()
            for h in range(HG):
                hb = h & 1
                if h + 1 < HG:
                    compute_widx(h + 1, (h + 1) & 1)
                    for cp in gather_descs((h + 1) & 1):
                        cp.start()
                for cp in gather_descs(hb):
                    cp.wait()
                if h >= 2:
                    out_desc(0, 0, hb).wait()
                transpose_rows(h, hb)
                out_desc(h0 + h, b0, hb).start()
            out_desc(0, 0, 0).wait()
            out_desc(0, 0, 1).wait()

        def body(t, carry):
            u = wid + NW * t

            @pl.when(u < n_units)
            def _():
                unit(u)

            return carry

        lax.fori_loop(0, per_tile, body, 0)

    return g_kernel


def kernel(x, action_emb_weight):
    V, D = action_emb_weight.shape
    NB, H = x.shape
    wT = action_emb_weight.T
    xT = x.T.astype(jnp.int32)
    n_tail = V % 128
    tail = action_emb_weight[V - n_tail :, :].reshape(n_tail // 2, 2 * D)
    wp = _make_transpose_kernel(V, D)(wT, tail)
    out_T = _make_gather_kernel(V, D, NB, H)(wp, xT)
    return jnp.transpose(out_T, (2, 0, 1))


# speedup vs baseline: 1.3222x; 1.3222x over previous
import functools

import jax
import jax.numpy as jnp
from jax import lax
from jax.experimental import pallas as pl
from jax.experimental.pallas import tpu as pltpu
from jax.experimental.pallas import tpu_sc as plsc

EMBED_DIM = 64


@functools.cache
def _make_gather(V: int, B: int, D: int):
    info = plsc.get_sparse_core_info()
    NC, NS = info.num_cores, info.num_subcores
    NW = NC * NS
    assert B % NW == 0 and D % info.num_lanes == 0
    b_per_w = B // NW
    CHUNK = 512
    SUB = 128
    n_sub = CHUNK // SUB
    assert b_per_w % (2 * CHUNK) == 0
    n_pairs = b_per_w // (2 * CHUNK)

    mesh = plsc.VectorSubcoreMesh(core_axis_name="c", subcore_axis_name="s")

    @functools.partial(
        pl.kernel,
        mesh=mesh,
        compiler_params=pltpu.CompilerParams(use_tc_tiling_on_sc=False),
        out_type=jax.ShapeDtypeStruct((B, D), jnp.float32),
        scratch_types=[
            pltpu.VMEM((b_per_w,), jnp.int32),
            pltpu.VMEM((2, CHUNK, D), jnp.float32),
            pltpu.SemaphoreType.DMA,
            pltpu.SemaphoreType.DMA,
            pltpu.SemaphoreType.DMA,
            pltpu.SemaphoreType.DMA,
        ],
    )
    def gather_kernel(table_hbm, idx_hbm, out_hbm, idx_v, rows_v, g0, g1, w0, w1):
        wid = lax.axis_index("s") * NC + lax.axis_index("c")
        base0 = wid * b_per_w
        gsems = (g0, g1)
        wsems = (w0, w1)

        pltpu.sync_copy(idx_hbm.at[pl.ds(base0, b_per_w)], idx_v)

        def fire_gathers(j, b):
            return [
                pltpu.async_copy(
                    table_hbm.at[idx_v.at[pl.ds(j * CHUNK + k * SUB, SUB)]],
                    rows_v.at[b, pl.ds(k * SUB, SUB)],
                    gsems[b],
                )
                for k in range(n_sub)
            ]

        def start_write(j, b):
            pltpu.make_async_copy(
                rows_v.at[b], out_hbm.at[pl.ds(base0 + j * CHUNK, CHUNK)], wsems[b]
            ).start()

        def wait_write(b):
            pltpu.make_async_copy(
                rows_v.at[b], out_hbm.at[pl.ds(base0, CHUNK)], wsems[b]
            ).wait()

        def body(t, carry):
            j0 = 2 * t
            j1 = 2 * t + 1

            @pl.when(t > 0)
            def _():
                wait_write(0)

            cps0 = fire_gathers(j0, 0)

            @pl.when(t > 0)
            def _():
                wait_write(1)

            cps1 = fire_gathers(j1, 1)
            for cp in cps0:
                cp.wait()
            start_write(j0, 0)
            for cp in cps1:
                cp.wait()
            start_write(j1, 1)
            return carry

        lax.fori_loop(0, n_pairs, body, 0)
        wait_write(0)
        wait_write(1)

    return gather_kernel


def kernel(x, action_emb_weight):
    V, D = action_emb_weight.shape
    idx = x.reshape(-1).astype(jnp.int32)
    out = _make_gather(V, idx.shape[0], D)(action_emb_weight, idx)
    return out.reshape(x.shape + (D,))
